# Initial kernel scaffold; baseline (speedup 1.0000x reference)
import jax
import jax.numpy as jnp
from jax.experimental import pallas as pl


def kernel(x, edge_index, W_mu, b_mu, W_logstd, b_logstd):
    raise NotImplementedError("write your pallas kernel here")


# speedup vs baseline: 26.0587x; 26.0587x over previous
import functools

import jax
import jax.numpy as jnp
from jax import lax
from jax.experimental import pallas as pl
from jax.experimental.pallas import tpu as pltpu
from jax.experimental.pallas import tpu_sc as plsc

N = 10000
E = 320000
D = 128
NPAD = 10240
CHUNK = 128
NUM_CHUNKS = E // CHUNK
TILES = 32
CHUNK_ITERS = -(-NUM_CHUNKS // TILES)
RPT = NPAD // 16
BLK = 512
GRID = -(-NPAD // BLK)


def _vmesh():
    return plsc.VectorSubcoreMesh(core_axis_name="c", subcore_axis_name="s")


def _sc_degree(col):
    @functools.partial(
        pl.kernel,
        out_type=jax.ShapeDtypeStruct((2 * NPAD, 16), jnp.float32),
        mesh=_vmesh(),
        scratch_types=[
            pltpu.VMEM_SHARED((NPAD, 16), jnp.float32),
            pltpu.VMEM((128, 16), jnp.float32),
            pltpu.VMEM((CHUNK, 16), jnp.float32),
            pltpu.VMEM((CHUNK,), jnp.int32),
        ],
    )
    def deg_kernel(col_hbm, out_hbm, dacc, zbuf, ones, coli):
        c = lax.axis_index("c")
        s = lax.axis_index("s")
        wid = s * 2 + c

        @pl.loop(0, 128)
        def _(i):
            zbuf[i] = jnp.zeros((16,), jnp.float32)
            ones[i] = jnp.ones((16,), jnp.float32)

        @pl.loop(0, RPT // 128)
        def _(k):
            pltpu.sync_copy(zbuf, dacc.at[pl.ds(s * RPT + k * 128, 128)])

        plsc.subcore_barrier()

        @pl.loop(0, CHUNK_ITERS)
        def _(i):
            j = wid + i * TILES

            @pl.when(j < NUM_CHUNKS)
            def _():
                pltpu.sync_copy(col_hbm.at[pl.ds(j * CHUNK, CHUNK)], coli)
                pltpu.sync_copy(ones, dacc.at[coli], add=True)

        plsc.subcore_barrier()
        pltpu.sync_copy(
            dacc.at[pl.ds(s * RPT, RPT)],
            out_hbm.at[pl.ds(c * NPAD + s * RPT, RPT)],
        )

    return deg_kernel(col)


def _sc_scatter(row, col, h2):
    @functools.partial(
        pl.kernel,
        out_type=jax.ShapeDtypeStruct((2 * NPAD, D), jnp.float32),
        mesh=_vmesh(),
        scratch_types=[
            pltpu.VMEM_SHARED((NPAD, D), jnp.float32),
            pltpu.VMEM((128, D), jnp.float32),
            pltpu.VMEM((CHUNK,), jnp.int32),
            pltpu.VMEM((CHUNK,), jnp.int32),
            pltpu.VMEM((CHUNK, D), jnp.float32),
            pltpu.SemaphoreType.DMA,
        ],
    )
    def scat_kernel(row_hbm, col_hbm, h2_hbm, out_hbm, acc, zbuf, rowi, coli, rows, sem):
        c = lax.axis_index("c")
        s = lax.axis_index("s")
        wid = s * 2 + c

        @pl.loop(0, 128)
        def _(i):
            @pl.loop(0, D, step=16)
            def _(j):
                zbuf[i, pl.ds(j, 16)] = jnp.zeros((16,), jnp.float32)

        @pl.loop(0, RPT // 128)
        def _(k):
            pltpu.sync_copy(zbuf, acc.at[pl.ds(s * RPT + k * 128, 128)])

        plsc.subcore_barrier()

        @pl.loop(0, CHUNK_ITERS)
        def _(i):
            j = wid + i * TILES

            @pl.when(j < NUM_CHUNKS)
            def _():
                pltpu.sync_copy(row_hbm.at[pl.ds(j * CHUNK, CHUNK)], rowi)
                pltpu.sync_copy(col_hbm.at[pl.ds(j * CHUNK, CHUNK)], coli)
                pltpu.async_copy(h2_hbm.at[rowi], rows, sem).wait()
                pltpu.sync_copy(rows, acc.at[coli], add=True)

        plsc.subcore_barrier()
        pltpu.sync_copy(
            acc.at[pl.ds(s * RPT, RPT)],
            out_hbm.at[pl.ds(c * NPAD + s * RPT, RPT)],
        )

    return scat_kernel(row, col, h2)


def _tc_h2(x, w, degp):
    def body(x_ref, w_ref, d0_ref, d1_ref, h2_ref):
        d = d0_ref[:, 0:1] + d1_ref[:, 0:1] + 1.0
        dis = lax.rsqrt(d)
        h = jnp.dot(x_ref[...], w_ref[...], preferred_element_type=jnp.float32)
        h2_ref[...] = h * dis

    return pl.pallas_call(
        body,
        grid=(GRID,),
        in_specs=[
            pl.BlockSpec((BLK, 128), lambda i: (i, 0)),
            pl.BlockSpec((128, 128), lambda i: (0, 0)),
            pl.BlockSpec((BLK, 16), lambda i: (i, 0)),
            pl.BlockSpec((BLK, 16), lambda i: (i + GRID, 0)),
        ],
        out_specs=pl.BlockSpec((BLK, 128), lambda i: (i, 0)),
        out_shape=jax.ShapeDtypeStruct((N, 128), jnp.float32),
    )(x, w, degp, degp)


def _tc_final(tmp, h2, degp, bmu, bls):
    def body(t0, t1, h2r, d0, d1, bm, bl, mu_ref, ls_ref):
        d = d0[:, 0:1] + d1[:, 0:1] + 1.0
        dis = lax.rsqrt(d)
        out = dis * (t0[...] + t1[...] + h2r[...])
        mu_ref[...] = out[:, :64] + bm[0:1, :]
        ls_ref[...] = out[:, 64:] + bl[0:1, :]

    return pl.pallas_call(
        body,
        grid=(GRID,),
        in_specs=[
            pl.BlockSpec((BLK, 128), lambda i: (i, 0)),
            pl.BlockSpec((BLK, 128), lambda i: (i + GRID, 0)),
            pl.BlockSpec((BLK, 128), lambda i: (i, 0)),
            pl.BlockSpec((BLK, 16), lambda i: (i, 0)),
            pl.BlockSpec((BLK, 16), lambda i: (i + GRID, 0)),
            pl.BlockSpec((8, 64), lambda i: (0, 0)),
            pl.BlockSpec((8, 64), lambda i: (0, 0)),
        ],
        out_specs=[
            pl.BlockSpec((BLK, 64), lambda i: (i, 0)),
            pl.BlockSpec((BLK, 64), lambda i: (i, 0)),
        ],
        out_shape=[
            jax.ShapeDtypeStruct((N, 64), jnp.float32),
            jax.ShapeDtypeStruct((N, 64), jnp.float32),
        ],
    )(tmp, tmp, h2, degp, degp, bmu, bls)


def kernel(x, edge_index, W_mu, b_mu, W_logstd, b_logstd):
    w = jnp.concatenate([W_mu, W_logstd], axis=1)
    row = edge_index[0]
    col = edge_index[1]
    bmu = jnp.tile(b_mu[None, :], (8, 1))
    bls = jnp.tile(b_logstd[None, :], (8, 1))
    degp = _sc_degree(col)
    h2 = _tc_h2(x, w, degp)
    tmp = _sc_scatter(row, col, h2)
    mu, logstd = _tc_final(tmp, h2, degp, bmu, bls)
    return (mu, logstd)
